# baseline (speedup 1.0000x reference)
import functools

import jax
import jax.numpy as jnp
from jax import lax
from jax.experimental import pallas as pl
from jax.experimental.pallas import tpu as pltpu
from jax.experimental.pallas import tpu_sc as plsc

_B, _D = 16384, 768
_L = 16
_NC, _NS = 2, 16
_NW = _NC * _NS
_BPW = _B // _NW
_CH = 64
_NCH = _BPW // _CH
_NJ = _D // _L
_INV_D = 1.0 / _D


_GATHER_DNUMS = lax.GatherDimensionNumbers(
    offset_dims=(), collapsed_slice_dims=(0,), start_index_map=(0,))


def _lane_shuffle(v, idx):
    return lax.gather(v, idx[:, None], _GATHER_DNUMS, (1,),
                      mode=lax.GatherScatterMode.PROMISE_IN_BOUNDS)


def _lane_sum(v):
    lane = lax.iota(jnp.int32, _L)
    for sh in (8, 4, 2, 1):
        v = v + _lane_shuffle(v, lane ^ sh)
    return v


def _body(ids, table, gamma, beta, out, idx_v, rows_v, gamma_v, beta_v,
          sem0, sem1):
    wid = lax.axis_index("s") * _NC + lax.axis_index("c")
    base = wid * _BPW
    pltpu.sync_copy(ids.at[pl.ds(base, _BPW)], idx_v)
    pltpu.sync_copy(gamma, gamma_v)
    pltpu.sync_copy(beta, beta_v)
    sems = (sem0, sem1)

    def gather_copy(c, b):
        return pltpu.make_async_copy(
            table.at[idx_v.at[pl.ds(c * _CH, _CH)]], rows_v.at[b], sems[b])

    gather_copy(0, 0).start()
    gather_copy(1, 1).start()

    def chunk_body(i, carry):
        c0 = i * 2
        for b in range(2):
            c = c0 + b
            gather_copy(c, b).wait()
            rows = rows_v.at[b]

            def row_body(r, carry_r):
                s = jnp.zeros((_L,), jnp.float32)
                q = jnp.zeros((_L,), jnp.float32)
                for j in range(_NJ):
                    v = rows[r, pl.ds(j * _L, _L)]
                    s = s + v
                    q = q + v * v
                m = _lane_sum(s) * _INV_D
                vv = _lane_sum(q) * _INV_D - m * m + 1e-5
                iv = lax.bitcast_convert_type(vv, jnp.int32)
                y = lax.bitcast_convert_type(
                    jnp.int32(0x5F3759DF) - (iv >> 1), jnp.float32)
                for _ in range(3):
                    y = y * (1.5 - 0.5 * vv * y * y)
                for j in range(_NJ):
                    sl = pl.ds(j * _L, _L)
                    v = rows[r, sl]
                    rows[r, sl] = (v - m) * y * gamma_v[sl] + beta_v[sl]
                return carry_r

            lax.fori_loop(0, _CH, row_body, 0)
            pltpu.sync_copy(rows_v.at[b], out.at[pl.ds(base + c * _CH, _CH)])

            @pl.when(c + 2 < _NCH)
            def _():
                gather_copy(c + 2, b).start()
        return carry

    lax.fori_loop(0, _NCH // 2, chunk_body, 0)


_encode = functools.partial(
    pl.kernel,
    out_type=jax.ShapeDtypeStruct((_B, _D), jnp.float32),
    mesh=plsc.VectorSubcoreMesh(core_axis_name="c", subcore_axis_name="s",
                                num_cores=_NC, num_subcores=_NS),
    scratch_types=[
        pltpu.VMEM((_BPW,), jnp.int32),
        pltpu.VMEM((2, _CH, _D), jnp.float32),
        pltpu.VMEM((_D,), jnp.float32),
        pltpu.VMEM((_D,), jnp.float32),
        pltpu.SemaphoreType.DMA,
        pltpu.SemaphoreType.DMA,
    ],
)(_body)


def kernel(user_ids, table, gamma, beta):
    return _encode(user_ids.astype(jnp.int32), table, gamma, beta)


# speedup vs baseline: 2.0299x; 2.0299x over previous
import functools

import jax
import jax.numpy as jnp
from jax import lax
from jax.experimental import pallas as pl
from jax.experimental.pallas import tpu as pltpu
from jax.experimental.pallas import tpu_sc as plsc

_B, _D = 16384, 768
_L = 16
_NC, _NS = 2, 16
_NW = _NC * _NS
_BPW = _B // _NW
_CH = 64
_NCH = _BPW // _CH
_NJ = _D // _L
_G = 4
_INV_D = 1.0 / _D


_GATHER_DNUMS = lax.GatherDimensionNumbers(
    offset_dims=(), collapsed_slice_dims=(0,), start_index_map=(0,))


def _lane_shuffle(v, idx):
    return lax.gather(v, idx[:, None], _GATHER_DNUMS, (1,),
                      mode=lax.GatherScatterMode.PROMISE_IN_BOUNDS)


def _lane_sum(v):
    lane = lax.iota(jnp.int32, _L)
    for sh in (8, 4, 2, 1):
        v = v + _lane_shuffle(v, lane ^ sh)
    return v


def _body(ids, table, gamma, beta, out, idx_v, rows_v, gamma_v, beta_v,
          sem0, sem1):
    wid = lax.axis_index("s") * _NC + lax.axis_index("c")
    base = wid * _BPW
    pltpu.sync_copy(ids.at[pl.ds(base, _BPW)], idx_v)
    pltpu.sync_copy(gamma, gamma_v)
    pltpu.sync_copy(beta, beta_v)
    sems = (sem0, sem1)

    def gather_copy(c, b):
        return pltpu.make_async_copy(
            table.at[idx_v.at[pl.ds(c * _CH, _CH)]], rows_v.at[b], sems[b])

    gather_copy(0, 0).start()
    gather_copy(1, 1).start()

    def chunk_body(i, carry):
        c0 = i * 2
        for b in range(2):
            c = c0 + b
            gather_copy(c, b).wait()
            rows = rows_v.at[b]

            def row_body(rg, carry_r):
                r0 = rg * _G
                s0 = [jnp.zeros((_L,), jnp.float32) for _ in range(_G)]
                s1 = [jnp.zeros((_L,), jnp.float32) for _ in range(_G)]
                q0 = [jnp.zeros((_L,), jnp.float32) for _ in range(_G)]
                q1 = [jnp.zeros((_L,), jnp.float32) for _ in range(_G)]
                for j in range(_NJ):
                    sl = pl.ds(j * _L, _L)
                    for g in range(_G):
                        v = rows[r0 + g, sl]
                        if j % 2 == 0:
                            s0[g] = s0[g] + v
                            q0[g] = q0[g] + v * v
                        else:
                            s1[g] = s1[g] + v
                            q1[g] = q1[g] + v * v
                m, y = [], []
                for g in range(_G):
                    mg = _lane_sum(s0[g] + s1[g]) * _INV_D
                    vv = (_lane_sum(q0[g] + q1[g]) * _INV_D
                          - mg * mg + 1e-5)
                    iv = lax.bitcast_convert_type(vv, jnp.int32)
                    yg = lax.bitcast_convert_type(
                        jnp.int32(0x5F3759DF) - (iv >> 1), jnp.float32)
                    for _ in range(2):
                        yg = yg * (1.5 - 0.5 * vv * yg * yg)
                    m.append(mg)
                    y.append(yg)
                for j in range(_NJ):
                    sl = pl.ds(j * _L, _L)
                    gj = gamma_v[sl]
                    bj = beta_v[sl]
                    for g in range(_G):
                        v = rows[r0 + g, sl]
                        rows[r0 + g, sl] = (v - m[g]) * y[g] * gj + bj
                return carry_r

            lax.fori_loop(0, _CH // _G, row_body, 0)
            pltpu.sync_copy(rows_v.at[b], out.at[pl.ds(base + c * _CH, _CH)])

            @pl.when(c + 2 < _NCH)
            def _():
                gather_copy(c + 2, b).start()
        return carry

    lax.fori_loop(0, _NCH // 2, chunk_body, 0)


_encode = functools.partial(
    pl.kernel,
    out_type=jax.ShapeDtypeStruct((_B, _D), jnp.float32),
    mesh=plsc.VectorSubcoreMesh(core_axis_name="c", subcore_axis_name="s",
                                num_cores=_NC, num_subcores=_NS),
    scratch_types=[
        pltpu.VMEM((_BPW,), jnp.int32),
        pltpu.VMEM((2, _CH, _D), jnp.float32),
        pltpu.VMEM((_D,), jnp.float32),
        pltpu.VMEM((_D,), jnp.float32),
        pltpu.SemaphoreType.DMA,
        pltpu.SemaphoreType.DMA,
    ],
)(_body)


def kernel(user_ids, table, gamma, beta):
    return _encode(user_ids.astype(jnp.int32), table, gamma, beta)
